# baseline (speedup 1.0000x reference)
import functools

import jax
import jax.numpy as jnp
from jax import lax
from jax.experimental import pallas as pl
from jax.experimental.pallas import tpu as pltpu
from jax.experimental.pallas import tpu_sc as plsc

BATCH = 4096
SEQ_LEN = 200
PQ_M = 8
VALS_PER_DIM = 256
SUB_DIM = 16

N_TOK = BATCH * SEQ_LEN
NUM_ITEMS_PAD = 1000002
NUM_CORES = 2
NUM_SUBCORES = 16
NW = NUM_CORES * NUM_SUBCORES
TOK_PER_W = N_TOK // NW
CHUNK = 256
NCHUNK = TOK_PER_W // CHUNK
IDS_SLICES = CHUNK // 128
IDX_SLICES = PQ_M * CHUNK // 128


def _sc_body(ids_hbm, w0_hbm, w1_hbm, cent_hbm, out_hbm,
             cent_sp,
             ids_v0, ids_v1, codes_v0, codes_v1, cidx_v0, cidx_v1,
             out_v0, out_v1,
             sem_i0, sem_i1, sem_c0, sem_c1, sem_g0, sem_g1, sem_h0, sem_h1,
             sem_o0, sem_o1):
    ids_v = (ids_v0, ids_v1)
    codes_v = (codes_v0, codes_v1)
    cidx_v = (cidx_v0, cidx_v1)
    out_v = (out_v0, out_v1)
    sem_i = (sem_i0, sem_i1)
    sem_c = (sem_c0, sem_c1)
    sem_g = (sem_g0, sem_g1)
    sem_h = (sem_h0, sem_h1)
    sem_o = (sem_o0, sem_o1)

    wid = lax.axis_index("s") * NUM_CORES + lax.axis_index("c")
    iota = lax.iota(jnp.int32, 16)
    offv = (iota >> 3) + ((iota >> 2) & 1) * CHUNK
    shiftv = (iota & 3) * 8
    maddv = ((iota & 7) << 8) + (
        (lax.axis_index("s") & 3) * (PQ_M * VALS_PER_DIM))

    def ids_slice(g):
        return ids_hbm.at[pl.ds((wid * NCHUNK + g) * CHUNK, CHUNK)]

    def codes_copies(p):
        idx = ids_v[p]
        return (
            pltpu.make_async_copy(
                w0_hbm.at[idx], codes_v[p].at[pl.ds(0, CHUNK)], sem_c[p]),
            pltpu.make_async_copy(
                w1_hbm.at[idx], codes_v[p].at[pl.ds(CHUNK, CHUNK)], sem_c[p]),
        )

    def fire_codes(p):
        for cp in codes_copies(p):
            cp.start()

    def drain_codes(p):
        for cp in codes_copies(p):
            cp.wait()

    def unpack(p):
        def body(i, c):
            w = plsc.load_gather(codes_v[p], [2 * i + offv])
            cidx_v[p][pl.ds(i * 16, 16)] = ((w >> shiftv) & 255) + maddv
            return c

        lax.fori_loop(0, PQ_M * CHUNK // 16, body, 0)

    HALF = PQ_M * CHUNK // 2

    def cent_copy(p, half):
        sem = sem_g[p] if half == 0 else sem_h[p]
        return pltpu.make_async_copy(
            cent_sp.at[cidx_v[p].at[pl.ds(half * HALF, HALF)]],
            out_v[p].at[pl.ds(half * HALF, HALF)], sem)

    def out_copy(p, g, half):
        return pltpu.make_async_copy(
            out_v[p].at[pl.ds(half * HALF, HALF)],
            out_hbm.at[pl.ds((wid * NCHUNK + g) * CHUNK * PQ_M + half * HALF,
                             HALF)],
            sem_o[p])

    sid = lax.axis_index("s")
    for r in range(4):
        @pl.when(sid == r)
        def _stage_cent():
            pltpu.sync_copy(
                cent_hbm,
                cent_sp.at[pl.ds(r * PQ_M * VALS_PER_DIM, PQ_M * VALS_PER_DIM)])

    plsc.subcore_barrier()

    pltpu.sync_copy(ids_slice(0), ids_v[0])
    fire_codes(0)
    pltpu.make_async_copy(ids_slice(1), ids_v[1], sem_i[1]).start()

    def pair_body(h, carry):
        for p in (0, 1):
            g = 2 * h + p
            q = 1 - p

            @pl.when(g + 1 < NCHUNK)
            def _stage_next():
                pltpu.make_async_copy(
                    ids_slice(g + 1), ids_v[q], sem_i[q]).wait()
                fire_codes(q)

            @pl.when(g + 2 < NCHUNK)
            def _prefetch_ids():
                pltpu.make_async_copy(
                    ids_slice(g + 2), ids_v[p], sem_i[p]).start()

            drain_codes(p)
            unpack(p)

            @pl.when(g >= 2)
            def _free_out():
                out_copy(p, g - 2, 0).wait()
                out_copy(p, g - 2, 1).wait()

            cent_copy(p, 0).start()
            cent_copy(p, 1).start()
            cent_copy(p, 0).wait()
            out_copy(p, g, 0).start()
            cent_copy(p, 1).wait()
            out_copy(p, g, 1).start()
        return carry

    lax.fori_loop(0, NCHUNK // 2, pair_body, 0)

    out_copy(0, NCHUNK - 2, 0).wait()
    out_copy(0, NCHUNK - 2, 1).wait()
    out_copy(1, NCHUNK - 1, 0).wait()
    out_copy(1, NCHUNK - 1, 1).wait()


@functools.partial(jax.jit)
def kernel(input_ids, item_codes, centroids):
    ids = input_ids.reshape(N_TOK)
    c = item_codes.astype(jnp.int32)
    w0 = c[:, 0] | (c[:, 1] << 8) | (c[:, 2] << 16) | (c[:, 3] << 24)
    w1 = c[:, 4] | (c[:, 5] << 8) | (c[:, 6] << 16) | (c[:, 7] << 24)
    cent = centroids.reshape(PQ_M * VALS_PER_DIM, SUB_DIM)

    call = pl.kernel(
        _sc_body,
        out_type=jax.ShapeDtypeStruct((N_TOK * PQ_M, SUB_DIM), jnp.float32),
        mesh=plsc.VectorSubcoreMesh(core_axis_name="c", subcore_axis_name="s"),
        compiler_params=pltpu.CompilerParams(
            needs_layout_passes=False, use_tc_tiling_on_sc=False),
        scratch_types=(
            [pltpu.VMEM_SHARED((4 * PQ_M * VALS_PER_DIM, SUB_DIM), jnp.float32)]
            + [pltpu.VMEM((CHUNK,), jnp.int32)] * 2
            + [pltpu.VMEM((2 * CHUNK,), jnp.int32)] * 2
            + [pltpu.VMEM((PQ_M * CHUNK,), jnp.int32)] * 2
            + [pltpu.VMEM((PQ_M * CHUNK, SUB_DIM), jnp.float32)] * 2
            + [pltpu.SemaphoreType.DMA] * 10
        ),
    )
    out = call(ids, w0, w1, cent)
    return out.reshape(BATCH, SEQ_LEN, PQ_M * SUB_DIM)


# speedup vs baseline: 1.0097x; 1.0097x over previous
import functools

import jax
import jax.numpy as jnp
from jax import lax
from jax.experimental import pallas as pl
from jax.experimental.pallas import tpu as pltpu
from jax.experimental.pallas import tpu_sc as plsc

BATCH = 4096
SEQ_LEN = 200
PQ_M = 8
VALS_PER_DIM = 256
SUB_DIM = 16

N_TOK = BATCH * SEQ_LEN
NUM_ITEMS_PAD = 1000002
NUM_CORES = 2
NUM_SUBCORES = 16
NW = NUM_CORES * NUM_SUBCORES
TOK_PER_W = N_TOK // NW
CHUNK = 320
NCHUNK = TOK_PER_W // CHUNK
IDS_SLICES = CHUNK // 128
IDX_SLICES = PQ_M * CHUNK // 128


def _sc_body(ids_hbm, w0_hbm, w1_hbm, cent_hbm, out_hbm,
             cent_sp,
             ids_v0, ids_v1, codes_v0, codes_v1, cidx_v0, cidx_v1,
             out_v0, out_v1,
             sem_i0, sem_i1, sem_c0, sem_c1, sem_g0, sem_g1, sem_h0, sem_h1,
             sem_o0, sem_o1):
    ids_v = (ids_v0, ids_v1)
    codes_v = (codes_v0, codes_v1)
    cidx_v = (cidx_v0, cidx_v1)
    out_v = (out_v0, out_v1)
    sem_i = (sem_i0, sem_i1)
    sem_c = (sem_c0, sem_c1)
    sem_g = (sem_g0, sem_g1)
    sem_h = (sem_h0, sem_h1)
    sem_o = (sem_o0, sem_o1)

    wid = lax.axis_index("s") * NUM_CORES + lax.axis_index("c")
    iota = lax.iota(jnp.int32, 16)
    offv = (iota >> 3) + ((iota >> 2) & 1) * CHUNK
    shiftv = (iota & 3) * 8
    maddv = ((iota & 7) << 8) + (
        (lax.axis_index("s") & 3) * (PQ_M * VALS_PER_DIM))

    def ids_slice(g):
        return ids_hbm.at[pl.ds((wid * NCHUNK + g) * CHUNK, CHUNK)]

    def codes_copies(p):
        idx = ids_v[p]
        return (
            pltpu.make_async_copy(
                w0_hbm.at[idx], codes_v[p].at[pl.ds(0, CHUNK)], sem_c[p]),
            pltpu.make_async_copy(
                w1_hbm.at[idx], codes_v[p].at[pl.ds(CHUNK, CHUNK)], sem_c[p]),
        )

    def fire_codes(p):
        for cp in codes_copies(p):
            cp.start()

    def drain_codes(p):
        for cp in codes_copies(p):
            cp.wait()

    def unpack(p):
        def body(i, c):
            w = plsc.load_gather(codes_v[p], [2 * i + offv])
            cidx_v[p][pl.ds(i * 16, 16)] = ((w >> shiftv) & 255) + maddv
            return c

        lax.fori_loop(0, PQ_M * CHUNK // 16, body, 0)

    HALF = PQ_M * CHUNK // 2

    def cent_copy(p, half):
        sem = sem_g[p] if half == 0 else sem_h[p]
        return pltpu.make_async_copy(
            cent_sp.at[cidx_v[p].at[pl.ds(half * HALF, HALF)]],
            out_v[p].at[pl.ds(half * HALF, HALF)], sem)

    def out_copy(p, g, half):
        return pltpu.make_async_copy(
            out_v[p].at[pl.ds(half * HALF, HALF)],
            out_hbm.at[pl.ds((wid * NCHUNK + g) * CHUNK * PQ_M + half * HALF,
                             HALF)],
            sem_o[p])

    sid = lax.axis_index("s")
    for r in range(4):
        @pl.when(sid == r)
        def _stage_cent():
            pltpu.sync_copy(
                cent_hbm,
                cent_sp.at[pl.ds(r * PQ_M * VALS_PER_DIM, PQ_M * VALS_PER_DIM)])

    plsc.subcore_barrier()

    pltpu.sync_copy(ids_slice(0), ids_v[0])
    fire_codes(0)
    pltpu.make_async_copy(ids_slice(1), ids_v[1], sem_i[1]).start()

    def pair_body(h, carry):
        for p in (0, 1):
            g = 2 * h + p
            q = 1 - p

            @pl.when(g + 1 < NCHUNK)
            def _stage_next():
                pltpu.make_async_copy(
                    ids_slice(g + 1), ids_v[q], sem_i[q]).wait()
                fire_codes(q)

            @pl.when(g + 2 < NCHUNK)
            def _prefetch_ids():
                pltpu.make_async_copy(
                    ids_slice(g + 2), ids_v[p], sem_i[p]).start()

            drain_codes(p)
            unpack(p)

            @pl.when(g >= 2)
            def _free_out():
                out_copy(p, g - 2, 0).wait()
                out_copy(p, g - 2, 1).wait()

            cent_copy(p, 0).start()
            cent_copy(p, 1).start()
            cent_copy(p, 0).wait()
            out_copy(p, g, 0).start()
            cent_copy(p, 1).wait()
            out_copy(p, g, 1).start()
        return carry

    lax.fori_loop(0, NCHUNK // 2, pair_body, 0)

    out_copy(0, NCHUNK - 2, 0).wait()
    out_copy(0, NCHUNK - 2, 1).wait()
    out_copy(1, NCHUNK - 1, 0).wait()
    out_copy(1, NCHUNK - 1, 1).wait()


@functools.partial(jax.jit)
def kernel(input_ids, item_codes, centroids):
    ids = input_ids.reshape(N_TOK)
    c = item_codes.astype(jnp.int32)
    w0 = c[:, 0] | (c[:, 1] << 8) | (c[:, 2] << 16) | (c[:, 3] << 24)
    w1 = c[:, 4] | (c[:, 5] << 8) | (c[:, 6] << 16) | (c[:, 7] << 24)
    cent = centroids.reshape(PQ_M * VALS_PER_DIM, SUB_DIM)

    call = pl.kernel(
        _sc_body,
        out_type=jax.ShapeDtypeStruct((N_TOK * PQ_M, SUB_DIM), jnp.float32),
        mesh=plsc.VectorSubcoreMesh(core_axis_name="c", subcore_axis_name="s"),
        compiler_params=pltpu.CompilerParams(
            needs_layout_passes=False, use_tc_tiling_on_sc=False),
        scratch_types=(
            [pltpu.VMEM_SHARED((4 * PQ_M * VALS_PER_DIM, SUB_DIM), jnp.float32)]
            + [pltpu.VMEM((CHUNK,), jnp.int32)] * 2
            + [pltpu.VMEM((2 * CHUNK,), jnp.int32)] * 2
            + [pltpu.VMEM((PQ_M * CHUNK,), jnp.int32)] * 2
            + [pltpu.VMEM((PQ_M * CHUNK, SUB_DIM), jnp.float32)] * 2
            + [pltpu.SemaphoreType.DMA] * 10
        ),
    )
    out = call(ids, w0, w1, cent)
    return out.reshape(BATCH, SEQ_LEN, PQ_M * SUB_DIM)
